# baseline (speedup 1.0000x reference)
import functools

import jax
import jax.numpy as jnp
from jax import lax
from jax.experimental import pallas as pl
from jax.experimental.pallas import tpu as pltpu
from jax.experimental.pallas import tpu_sc as plsc

_BATCH = 16384
_HIST = 50
_N = _BATCH * _HIST
_NW = 32
_PER_W = _N // _NW


def _make_gather():
    mesh = plsc.VectorSubcoreMesh(core_axis_name="c", subcore_axis_name="s")

    @functools.partial(
        pl.kernel,
        mesh=mesh,
        out_type=jax.ShapeDtypeStruct((_N,), jnp.float32),
        scratch_types=[
            pltpu.VMEM((_PER_W,), jnp.int32),
            pltpu.VMEM((_PER_W,), jnp.float32),
            pltpu.SemaphoreType.DMA,
        ],
    )
    def gather_kernel(x_hbm, tbl_hbm, out_hbm, idx_v, val_v, sem):
        wid = lax.axis_index("s") * 2 + lax.axis_index("c")
        base = wid * _PER_W
        pltpu.sync_copy(x_hbm.at[pl.ds(base, _PER_W)], idx_v)
        pltpu.async_copy(tbl_hbm.at[idx_v], val_v, sem).wait()
        pltpu.sync_copy(val_v, out_hbm.at[pl.ds(base, _PER_W)])

    return gather_kernel


def kernel(x, item_b):
    x32 = x.reshape(_N).astype(jnp.int32)
    out = _make_gather()(x32, item_b)
    return out.reshape(_BATCH, _HIST)


# speedup vs baseline: 1.2261x; 1.2261x over previous
import functools

import jax
import jax.numpy as jnp
from jax import lax
from jax.experimental import pallas as pl
from jax.experimental.pallas import tpu as pltpu
from jax.experimental.pallas import tpu_sc as plsc

_BATCH = 16384
_HIST = 50
_N = _BATCH * _HIST
_NW = 32
_PER_W = _N // _NW
_VOCAB = 1000000
_VOCAB_PER_S = _VOCAB // 16
_STRIPE = _VOCAB_PER_S + 4
_CHUNK = 7816
_NCHUNK = 8
_SIZES = [_CHUNK] * 7 + [_STRIPE - 7 * _CHUNK]
_OFFS = [k * _CHUNK for k in range(_NCHUNK)]


def _make_gather():
    mesh = plsc.VectorSubcoreMesh(core_axis_name="c", subcore_axis_name="s")

    @functools.partial(
        pl.kernel,
        mesh=mesh,
        out_type=jax.ShapeDtypeStruct((_N,), jnp.float32),
        scratch_types=[
            pltpu.VMEM((_PER_W,), jnp.int32),
            pltpu.VMEM((_PER_W,), jnp.float32),
            pltpu.VMEM((_CHUNK,), jnp.float32),
            pltpu.VMEM((_CHUNK,), jnp.float32),
            pltpu.VMEM_SHARED((_VOCAB,), jnp.float32),
            pltpu.SemaphoreType.DMA,
            pltpu.SemaphoreType.DMA,
        ],
    )
    def gather_kernel(x_hbm, tbl_hbm, out_hbm, idx_v, val_v, buf0, buf1,
                      tbl_s, sem, sem2):
        cid = lax.axis_index("c")
        sid = lax.axis_index("s")
        base = (sid * 2 + cid) * _PER_W

        s0 = sid * _VOCAB_PER_S
        start = pl.multiple_of(s0 - lax.rem(s0, 8), 8)
        bufs = (buf0, buf1)
        copies = [
            pltpu.async_copy(
                tbl_hbm.at[pl.ds(start + _OFFS[0], _SIZES[0])],
                buf0.at[pl.ds(0, _SIZES[0])], sem2),
            pltpu.async_copy(
                tbl_hbm.at[pl.ds(start + _OFFS[1], _SIZES[1])],
                buf1.at[pl.ds(0, _SIZES[1])], sem2),
        ]
        pltpu.sync_copy(x_hbm.at[pl.ds(base, _PER_W)], idx_v)
        for k in range(_NCHUNK):
            copies[k % 2].wait()
            pltpu.sync_copy(
                bufs[k % 2].at[pl.ds(0, _SIZES[k])],
                tbl_s.at[pl.ds(start + _OFFS[k], _SIZES[k])])
            if k + 2 < _NCHUNK:
                copies[k % 2] = pltpu.async_copy(
                    tbl_hbm.at[pl.ds(start + _OFFS[k + 2], _SIZES[k + 2])],
                    bufs[k % 2].at[pl.ds(0, _SIZES[k + 2])], sem2)
        plsc.subcore_barrier()
        pltpu.async_copy(tbl_s.at[idx_v], val_v, sem).wait()
        pltpu.sync_copy(val_v, out_hbm.at[pl.ds(base, _PER_W)])

    return gather_kernel


def kernel(x, item_b):
    x32 = x.reshape(_N).astype(jnp.int32)
    out = _make_gather()(x32, item_b)
    return out.reshape(_BATCH, _HIST)


# speedup vs baseline: 1.8178x; 1.4826x over previous
import functools

import jax
import jax.numpy as jnp
from jax import lax
from jax.experimental import pallas as pl
from jax.experimental.pallas import tpu as pltpu
from jax.experimental.pallas import tpu_sc as plsc

_BATCH = 16384
_HIST = 50
_NW = 32
_ROWS_W = _BATCH // _NW
_BLK = 64
_NBLK = _ROWS_W // _BLK
_VOCAB = 1000000
_VOCAB_PER_S = _VOCAB // 16
_STRIPE = _VOCAB_PER_S + 4
_CHUNK = 7816
_NCHUNK = 8
_SIZES = [_CHUNK] * 7 + [_STRIPE - 7 * _CHUNK]
_OFFS = [k * _CHUNK for k in range(_NCHUNK)]


def _make_gather():
    mesh = plsc.VectorSubcoreMesh(core_axis_name="c", subcore_axis_name="s")

    @functools.partial(
        pl.kernel,
        mesh=mesh,
        out_type=jax.ShapeDtypeStruct((_BATCH, _HIST), jnp.float32),
        scratch_types=[
            pltpu.VMEM((_BLK, _HIST), jnp.int32),
            pltpu.VMEM((_BLK, _HIST), jnp.int32),
            pltpu.VMEM((_BLK, _HIST), jnp.float32),
            pltpu.VMEM((_BLK, _HIST), jnp.float32),
            pltpu.VMEM((_CHUNK,), jnp.float32),
            pltpu.VMEM((_CHUNK,), jnp.float32),
            pltpu.VMEM_SHARED((_VOCAB,), jnp.float32),
            pltpu.SemaphoreType.DMA,
            pltpu.SemaphoreType.DMA,
            pltpu.SemaphoreType.DMA,
            pltpu.SemaphoreType.DMA,
        ],
    )
    def gather_kernel(x_hbm, tbl_hbm, out_hbm, idx0, idx1, val0, val1,
                      buf0, buf1, tbl_s, semg, sem2, semi, semw):
        cid = lax.axis_index("c")
        sid = lax.axis_index("s")
        base = (sid * 2 + cid) * _ROWS_W

        s0 = sid * _VOCAB_PER_S
        start = pl.multiple_of(s0 - lax.rem(s0, 8), 8)
        bufs = (buf0, buf1)
        copies = [
            pltpu.async_copy(
                tbl_hbm.at[pl.ds(start + _OFFS[0], _SIZES[0])],
                buf0.at[pl.ds(0, _SIZES[0])], sem2),
            pltpu.async_copy(
                tbl_hbm.at[pl.ds(start + _OFFS[1], _SIZES[1])],
                buf1.at[pl.ds(0, _SIZES[1])], sem2),
        ]
        idxb = (idx0, idx1)
        valb = (val0, val1)
        iload = [
            pltpu.async_copy(x_hbm.at[pl.ds(base, _BLK)], idx0, semi),
            pltpu.async_copy(x_hbm.at[pl.ds(base + _BLK, _BLK)], idx1, semi),
        ]
        for k in range(_NCHUNK):
            copies[k % 2].wait()
            pltpu.sync_copy(
                bufs[k % 2].at[pl.ds(0, _SIZES[k])],
                tbl_s.at[pl.ds(start + _OFFS[k], _SIZES[k])])
            if k + 2 < _NCHUNK:
                copies[k % 2] = pltpu.async_copy(
                    tbl_hbm.at[pl.ds(start + _OFFS[k + 2], _SIZES[k + 2])],
                    bufs[k % 2].at[pl.ds(0, _SIZES[k + 2])], sem2)
        plsc.subcore_barrier()

        wback = [None, None]
        for b in range(_NBLK):
            iv = idxb[b % 2]
            vv = valb[b % 2]
            iload[b % 2].wait()
            if wback[b % 2] is not None:
                wback[b % 2].wait()

            def fire(r, carry, iv=iv, vv=vv):
                pltpu.async_copy(tbl_s.at[iv.at[r]], vv.at[r], semg)
                return carry

            lax.fori_loop(0, _BLK, fire, 0)

            def drain(r, carry, iv=iv, vv=vv):
                pltpu.make_async_copy(
                    tbl_s.at[iv.at[r]], vv.at[r], semg).wait()
                return carry

            lax.fori_loop(0, _BLK, drain, 0)
            if b + 2 < _NBLK:
                iload[b % 2] = pltpu.async_copy(
                    x_hbm.at[pl.ds(base + (b + 2) * _BLK, _BLK)], iv, semi)
            wback[b % 2] = pltpu.async_copy(
                vv, out_hbm.at[pl.ds(base + b * _BLK, _BLK)], semw)
        wback[0].wait()
        wback[1].wait()

    return gather_kernel


def kernel(x, item_b):
    return _make_gather()(x.astype(jnp.int32), item_b)


# speedup vs baseline: 1.8189x; 1.0006x over previous
import functools

import jax
import jax.numpy as jnp
from jax import lax
from jax.experimental import pallas as pl
from jax.experimental.pallas import tpu as pltpu
from jax.experimental.pallas import tpu_sc as plsc

_BATCH = 16384
_HIST = 50
_NW = 32
_ROWS_W = _BATCH // _NW
_BLK = 64
_NBLK = _ROWS_W // _BLK
_VOCAB = 1000000
_VOCAB_PER_S = _VOCAB // 16
_STRIPE = _VOCAB_PER_S + 4
_CHUNK = 7816
_NCHUNK = 8
_SIZES = [_CHUNK] * 7 + [_STRIPE - 7 * _CHUNK]
_OFFS = [k * _CHUNK for k in range(_NCHUNK)]


def _make_gather():
    mesh = plsc.VectorSubcoreMesh(core_axis_name="c", subcore_axis_name="s")

    @functools.partial(
        pl.kernel,
        mesh=mesh,
        compiler_params=pltpu.CompilerParams(use_tc_tiling_on_sc=True),
        out_type=jax.ShapeDtypeStruct((_BATCH, _HIST), jnp.float32),
        scratch_types=[
            pltpu.VMEM((_BLK, _HIST), jnp.int32),
            pltpu.VMEM((_BLK, _HIST), jnp.int32),
            pltpu.VMEM((_BLK, _HIST), jnp.float32),
            pltpu.VMEM((_BLK, _HIST), jnp.float32),
            pltpu.VMEM((_CHUNK,), jnp.float32),
            pltpu.VMEM((_CHUNK,), jnp.float32),
            pltpu.VMEM_SHARED((_VOCAB,), jnp.float32),
            pltpu.SemaphoreType.DMA,
            pltpu.SemaphoreType.DMA,
            pltpu.SemaphoreType.DMA,
            pltpu.SemaphoreType.DMA,
        ],
    )
    def gather_kernel(x_hbm, tbl_hbm, out_hbm, idx0, idx1, val0, val1,
                      buf0, buf1, tbl_s, semg, sem2, semi, semw):
        cid = lax.axis_index("c")
        sid = lax.axis_index("s")
        base = (sid * 2 + cid) * _ROWS_W

        s0 = sid * _VOCAB_PER_S
        start = pl.multiple_of(s0 - lax.rem(s0, 8), 8)
        bufs = (buf0, buf1)
        copies = [
            pltpu.async_copy(
                tbl_hbm.at[pl.ds(start + _OFFS[0], _SIZES[0])],
                buf0.at[pl.ds(0, _SIZES[0])], sem2),
            pltpu.async_copy(
                tbl_hbm.at[pl.ds(start + _OFFS[1], _SIZES[1])],
                buf1.at[pl.ds(0, _SIZES[1])], sem2),
        ]
        idxb = (idx0, idx1)
        valb = (val0, val1)
        iload = [
            pltpu.async_copy(x_hbm.at[pl.ds(base, _BLK)], idx0, semi),
            pltpu.async_copy(x_hbm.at[pl.ds(base + _BLK, _BLK)], idx1, semi),
        ]
        for k in range(_NCHUNK):
            copies[k % 2].wait()
            pltpu.sync_copy(
                bufs[k % 2].at[pl.ds(0, _SIZES[k])],
                tbl_s.at[pl.ds(start + _OFFS[k], _SIZES[k])])
            if k + 2 < _NCHUNK:
                copies[k % 2] = pltpu.async_copy(
                    tbl_hbm.at[pl.ds(start + _OFFS[k + 2], _SIZES[k + 2])],
                    bufs[k % 2].at[pl.ds(0, _SIZES[k + 2])], sem2)
        plsc.subcore_barrier()

        wback = [None, None]
        for b in range(_NBLK):
            iv = idxb[b % 2]
            vv = valb[b % 2]
            iload[b % 2].wait()
            if wback[b % 2] is not None:
                wback[b % 2].wait()

            def fire(r, carry, iv=iv, vv=vv):
                pltpu.async_copy(tbl_s.at[iv.at[r]], vv.at[r], semg)
                return carry

            lax.fori_loop(0, _BLK, fire, 0)

            def drain(r, carry, iv=iv, vv=vv):
                pltpu.make_async_copy(
                    tbl_s.at[iv.at[r]], vv.at[r], semg).wait()
                return carry

            lax.fori_loop(0, _BLK, drain, 0)
            if b + 2 < _NBLK:
                iload[b % 2] = pltpu.async_copy(
                    x_hbm.at[pl.ds(base + (b + 2) * _BLK, _BLK)], iv, semi)
            wback[b % 2] = pltpu.async_copy(
                vv, out_hbm.at[pl.ds(base + b * _BLK, _BLK)], semw)
        wback[0].wait()
        wback[1].wait()

    return gather_kernel


def kernel(x, item_b):
    return _make_gather()(x.astype(jnp.int32), item_b)


# speedup vs baseline: 2.4985x; 1.3736x over previous
import functools

import jax
import jax.numpy as jnp
from jax import lax
from jax.experimental import pallas as pl
from jax.experimental.pallas import tpu as pltpu
from jax.experimental.pallas import tpu_sc as plsc

_BATCH = 16384
_HIST = 50
_NW = 32
_COLS_W = _BATCH // _NW
_VOCAB = 1000000
_VOCAB_PER_S = _VOCAB // 16
_STRIPE = _VOCAB_PER_S + 4
_CHUNK = 7816
_NCHUNK = 8
_SIZES = [_CHUNK] * 7 + [_STRIPE - 7 * _CHUNK]
_OFFS = [k * _CHUNK for k in range(_NCHUNK)]


def _make_gather():
    mesh = plsc.VectorSubcoreMesh(core_axis_name="c", subcore_axis_name="s")

    @functools.partial(
        pl.kernel,
        mesh=mesh,
        out_type=jax.ShapeDtypeStruct((_HIST, _BATCH), jnp.float32),
        scratch_types=[
            pltpu.VMEM((_HIST * _COLS_W,), jnp.int32),
            pltpu.VMEM((_HIST * _COLS_W,), jnp.float32),
            pltpu.VMEM((_CHUNK,), jnp.float32),
            pltpu.VMEM((_CHUNK,), jnp.float32),
            pltpu.VMEM_SHARED((_VOCAB,), jnp.float32),
            pltpu.SemaphoreType.DMA,
            pltpu.SemaphoreType.DMA,
            pltpu.SemaphoreType.DMA,
        ],
    )
    def gather_kernel(x_hbm, tbl_hbm, out_hbm, idx_v, val_v, buf0, buf1,
                      tbl_s, sem, sem2, sem3):
        cid = lax.axis_index("c")
        sid = lax.axis_index("s")
        col0 = (sid * 2 + cid) * _COLS_W

        s0 = sid * _VOCAB_PER_S
        start = pl.multiple_of(s0 - lax.rem(s0, 8), 8)
        bufs = (buf0, buf1)
        copies = [
            pltpu.async_copy(
                tbl_hbm.at[pl.ds(start + _OFFS[0], _SIZES[0])],
                buf0.at[pl.ds(0, _SIZES[0])], sem2),
            pltpu.async_copy(
                tbl_hbm.at[pl.ds(start + _OFFS[1], _SIZES[1])],
                buf1.at[pl.ds(0, _SIZES[1])], sem2),
        ]
        def iload(r, carry):
            pltpu.async_copy(
                x_hbm.at[r, pl.ds(col0, _COLS_W)],
                idx_v.at[pl.ds(r * _COLS_W, _COLS_W)], sem3)
            return carry

        lax.fori_loop(0, _HIST, iload, 0)
        for k in range(_NCHUNK):
            copies[k % 2].wait()
            pltpu.sync_copy(
                bufs[k % 2].at[pl.ds(0, _SIZES[k])],
                tbl_s.at[pl.ds(start + _OFFS[k], _SIZES[k])])
            if k + 2 < _NCHUNK:
                copies[k % 2] = pltpu.async_copy(
                    tbl_hbm.at[pl.ds(start + _OFFS[k + 2], _SIZES[k + 2])],
                    bufs[k % 2].at[pl.ds(0, _SIZES[k + 2])], sem2)
        def idrain(r, carry):
            pltpu.make_async_copy(
                x_hbm.at[r, pl.ds(col0, _COLS_W)],
                idx_v.at[pl.ds(r * _COLS_W, _COLS_W)], sem3).wait()
            return carry

        lax.fori_loop(0, _HIST, idrain, 0)
        plsc.subcore_barrier()
        pltpu.async_copy(tbl_s.at[idx_v], val_v, sem).wait()

        def wback(r, carry):
            pltpu.async_copy(
                val_v.at[pl.ds(r * _COLS_W, _COLS_W)],
                out_hbm.at[r, pl.ds(col0, _COLS_W)], sem3)
            return carry

        lax.fori_loop(0, _HIST, wback, 0)

        def wdrain(r, carry):
            pltpu.make_async_copy(
                val_v.at[pl.ds(r * _COLS_W, _COLS_W)],
                out_hbm.at[r, pl.ds(col0, _COLS_W)], sem3).wait()
            return carry

        lax.fori_loop(0, _HIST, wdrain, 0)

    return gather_kernel


def kernel(x, item_b):
    out_t = _make_gather()(x.T.astype(jnp.int32), item_b)
    return out_t.T
